# baseline (speedup 1.0000x reference)
import functools
import math

import jax
import jax.numpy as jnp
from jax import lax
from jax.experimental import pallas as pl
from jax.experimental.pallas import tpu as pltpu
from jax.experimental.pallas import tpu_sc as plsc

_N = 10000
_E = 320000
_D = 128
_A = 10
_R = 8
_H = 64
_AVG = 32.0

_NC = 2
_NS = 16
_NW = _NC * _NS
_CHUNK = 64
_CPW = 160
_GRP = 40
_NGRP = _CPW // _GRP
_EPW = _CPW * _CHUNK
_EPAD = _NW * _EPW
_NPAD = 10240
_RPT = _NPAD // _NS


def _node_up_body(nf_ref, w_ref, o_ref):
    o_ref[...] = jnp.dot(nf_ref[...], w_ref[...],
                         preferred_element_type=jnp.float32) * (1.0 / math.sqrt(_D))


def _node_up(node_feats, W_up):
    blk = 1000
    return pl.pallas_call(
        _node_up_body,
        grid=(_N // blk,),
        in_specs=[
            pl.BlockSpec((blk, _D), lambda i: (i, 0)),
            pl.BlockSpec((_D, _D), lambda i: (0, 0)),
        ],
        out_specs=pl.BlockSpec((blk, _D), lambda i: (i, 0)),
        out_shape=jax.ShapeDtypeStruct((_N, _D), jnp.float32),
    )(node_feats, W_up)


def _silu(h):
    return h * (0.5 + 0.5 * jnp.tanh(0.5 * h))


def _tdot(a, b):
    return lax.dot_general(a, b, (((0,), (0,)), ((), ())),
                           preferred_element_type=jnp.float32)


def _edge_mlp_body(eft_ref, eat_ref, w1_ref, w2_ref, w3_ref, w4_ref, o_ref):
    h = _tdot(w1_ref[...], eft_ref[...]) * (1.0 / math.sqrt(_R))
    h = _silu(h)
    h = _tdot(w2_ref[...], h) * (1.0 / math.sqrt(_H))
    h = _silu(h)
    h = _tdot(w3_ref[...], h) * (1.0 / math.sqrt(_H))
    h = _silu(h) * eat_ref[...]
    o_ref[...] = _tdot(h, w4_ref[...]) * (1.0 / math.sqrt(_H))


def _edge_mlp(eft, eat, W1, W2, W3, W4):
    blk = 4096
    return pl.pallas_call(
        _edge_mlp_body,
        grid=(_EPAD // blk,),
        in_specs=[
            pl.BlockSpec((_R, blk), lambda i: (0, i)),
            pl.BlockSpec((1, blk), lambda i: (0, i)),
            pl.BlockSpec((_R, _H), lambda i: (0, 0)),
            pl.BlockSpec((_H, _H), lambda i: (0, 0)),
            pl.BlockSpec((_H, _H), lambda i: (0, 0)),
            pl.BlockSpec((_H, _D), lambda i: (0, 0)),
        ],
        out_specs=pl.BlockSpec((blk, _D), lambda i: (i, 0)),
        out_shape=jax.ShapeDtypeStruct((_EPAD, _D), jnp.float32),
    )(eft, eat, W1, W2, W3, W4)


def _repack_body(ei_ref, o_ref, *, blk):
    i = pl.program_id(0)
    col = jax.lax.broadcasted_iota(jnp.int32, (2, blk), 1) + i * blk
    o_ref[...] = jnp.where(col < _E, ei_ref[...], 0)


def _repack_idx(edge_index):
    blk = _EPAD // 16
    return pl.pallas_call(
        functools.partial(_repack_body, blk=blk),
        grid=(16,),
        in_specs=[pl.BlockSpec((2, blk), lambda i: (0, i))],
        out_specs=pl.BlockSpec((2, blk), lambda i: (0, i)),
        out_shape=jax.ShapeDtypeStruct((2, _EPAD), jnp.int32),
    )(edge_index)


def _sc_gather_scatter(x, w, src2, dst2):
    mesh = plsc.VectorSubcoreMesh(core_axis_name="c", subcore_axis_name="s",
                                  num_cores=_NC, num_subcores=_NS)

    @functools.partial(
        pl.kernel,
        out_type=jax.ShapeDtypeStruct((_NC, _NPAD, _D), jnp.float32),
        mesh=mesh,
        scratch_types=[
            pltpu.VMEM((_GRP, _CHUNK), jnp.int32),
            pltpu.VMEM((_GRP, _CHUNK), jnp.int32),
            pltpu.VMEM((_CHUNK, _D), jnp.float32),
            pltpu.VMEM((_CHUNK, _D), jnp.float32),
            pltpu.VMEM((_CHUNK, _D), jnp.float32),
            pltpu.VMEM((_CHUNK, _D), jnp.float32),
            pltpu.VMEM_SHARED((_NPAD, _D), jnp.float32),
            pltpu.SemaphoreType.DMA,
            pltpu.SemaphoreType.DMA,
            pltpu.SemaphoreType.DMA,
            pltpu.SemaphoreType.DMA,
        ],
    )
    def k(x_hbm, w_hbm, src_hbm, dst_hbm, out_hbm,
          src_v, dst_v, rows0, rows1, wrow0, wrow1, acc,
          sg0, sg1, sw0, sw1):
        cid = lax.axis_index("c")
        sid = lax.axis_index("s")
        wid = cid * _NS + sid
        rows = (rows0, rows1)
        wrow = (wrow0, wrow1)
        sg = (sg0, sg1)
        sw = (sw0, sw1)

        def zrow(i, carry):
            for k8 in range(_D // 16):
                rows0[i, pl.ds(k8 * 16, 16)] = jnp.zeros((16,), jnp.float32)
            return carry
        lax.fori_loop(0, _CHUNK, zrow, 0)
        base_row = sid * _RPT
        for kk in range(_RPT // _CHUNK):
            pltpu.sync_copy(rows0, acc.at[pl.ds(base_row + kk * _CHUNK, _CHUNK)])
        plsc.subcore_barrier()

        def issue(h, c, b):
            ebase = wid * _EPW + h * _GRP * _CHUNK
            pltpu.async_copy(x_hbm.at[src_v.at[c]], rows[b], sg[b])
            pltpu.async_copy(w_hbm.at[pl.ds(ebase + c * _CHUNK, _CHUNK)],
                             wrow[b], sw[b])

        def wait(b):
            pltpu.make_async_copy(x_hbm.at[src_v.at[0]], rows[b], sg[b]).wait()
            pltpu.make_async_copy(w_hbm.at[pl.ds(0, _CHUNK)], wrow[b], sw[b]).wait()

        def process(c, b):
            @plsc.parallel_loop(0, _CHUNK, unroll=4)
            def mul(i):
                for k8 in range(_D // 16):
                    sl = pl.ds(k8 * 16, 16)
                    rows[b][i, sl] = rows[b][i, sl] * wrow[b][i, sl]
            pltpu.sync_copy(rows[b], acc.at[dst_v.at[c]], add=True)

        for h in range(_NGRP):
            pltpu.sync_copy(src_hbm.at[pl.ds(wid * _CPW + h * _GRP, _GRP)], src_v)
            pltpu.sync_copy(dst_hbm.at[pl.ds(wid * _CPW + h * _GRP, _GRP)], dst_v)
            issue(h, 0, 0)

            def pair(i, carry, h=h):
                c0 = 2 * i
                issue(h, c0 + 1, 1)
                wait(0)
                process(c0, 0)

                @pl.when(i < _GRP // 2 - 1)
                def _():
                    issue(h, c0 + 2, 0)
                wait(1)
                process(c0 + 1, 1)
                return carry
            lax.fori_loop(0, _GRP // 2, pair, 0)
        plsc.subcore_barrier()

        for kk in range(_RPT // _CHUNK):
            r0 = base_row + kk * _CHUNK
            pltpu.sync_copy(acc.at[pl.ds(r0, _CHUNK)], rows0)
            pltpu.sync_copy(rows0, out_hbm.at[cid].at[pl.ds(r0, _CHUNK)])

    return k(x, w, src2, dst2)


def _finish_body(p_ref, na_ref, wlin_ref, wskip_ref, o_ref):
    msg = (p_ref[0] + p_ref[1])
    msg = jnp.dot(msg, wlin_ref[...],
                  preferred_element_type=jnp.float32) * (1.0 / (math.sqrt(_D) * _AVG))
    acc = jnp.zeros_like(o_ref)
    for v in range(_A):
        acc = acc + na_ref[:, v:v + 1] * jnp.dot(
            msg, wskip_ref[v], preferred_element_type=jnp.float32)
    o_ref[...] = acc * (1.0 / math.sqrt(float(_D * _A)))


def _finish(partial, node_attrs, W_lin, W_skip_t):
    blk = 1000
    return pl.pallas_call(
        _finish_body,
        grid=(_N // blk,),
        in_specs=[
            pl.BlockSpec((_NC, blk, _D), lambda i: (0, i, 0)),
            pl.BlockSpec((blk, _A), lambda i: (i, 0)),
            pl.BlockSpec((_D, _D), lambda i: (0, 0)),
            pl.BlockSpec((_A, _D, _D), lambda i: (0, 0, 0)),
        ],
        out_specs=pl.BlockSpec((blk, _D), lambda i: (i, 0)),
        out_shape=jax.ShapeDtypeStruct((_N, _D), jnp.float32),
    )(partial, node_attrs, W_lin, W_skip_t)


def kernel(node_attrs, node_feats, edge_attrs, edge_feats, edge_index,
           W_up, W1, W2, W3, W4, W_lin, W_skip):
    pad = _EPAD - _E
    eft = jnp.pad(edge_feats.T, ((0, 0), (0, pad)))
    eat = jnp.pad(edge_attrs.T, ((0, 0), (0, pad)))
    ei = _repack_idx(edge_index)
    src2 = ei[0].reshape(_EPAD // _CHUNK, _CHUNK)
    dst2 = ei[1].reshape(_EPAD // _CHUNK, _CHUNK)

    x = _node_up(node_feats, W_up)
    w = _edge_mlp(eft, eat, W1, W2, W3, W4)
    partial = _sc_gather_scatter(x, w, src2, dst2)
    out = _finish(partial, node_attrs, W_lin, jnp.transpose(W_skip, (1, 0, 2)))
    return out.reshape(_N, _D, 1)


# speedup vs baseline: 1.0301x; 1.0301x over previous
import functools
import math

import jax
import jax.numpy as jnp
from jax import lax
from jax.experimental import pallas as pl
from jax.experimental.pallas import tpu as pltpu
from jax.experimental.pallas import tpu_sc as plsc

_N = 10000
_E = 320000
_D = 128
_A = 10
_R = 8
_H = 64
_AVG = 32.0

_NC = 2
_NS = 16
_NW = _NC * _NS
_CHUNK = 64
_CPW = 160
_GRP = 40
_NGRP = _CPW // _GRP
_EPW = _CPW * _CHUNK
_EPAD = _NW * _EPW
_NPAD = 10240
_RPT = _NPAD // _NS


def _node_up_body(nf_ref, w_ref, o_ref):
    o_ref[...] = jnp.dot(nf_ref[...], w_ref[...],
                         preferred_element_type=jnp.float32) * (1.0 / math.sqrt(_D))


def _node_up(node_feats, W_up):
    blk = 1000
    return pl.pallas_call(
        _node_up_body,
        grid=(_N // blk,),
        in_specs=[
            pl.BlockSpec((blk, _D), lambda i: (i, 0)),
            pl.BlockSpec((_D, _D), lambda i: (0, 0)),
        ],
        out_specs=pl.BlockSpec((blk, _D), lambda i: (i, 0)),
        out_shape=jax.ShapeDtypeStruct((_N, _D), jnp.float32),
    )(node_feats, W_up)


def _silu(h):
    return h * (0.5 + 0.5 * jnp.tanh(0.5 * h))


def _tdot(a, b):
    return lax.dot_general(a, b, (((0,), (0,)), ((), ())),
                           preferred_element_type=jnp.float32)


def _edge_mlp_body(eft_ref, eat_ref, w1_ref, w2_ref, w3_ref, w4_ref, o_ref):
    h = _tdot(w1_ref[...], eft_ref[...]) * (1.0 / math.sqrt(_R))
    h = _silu(h)
    h = _tdot(w2_ref[...], h) * (1.0 / math.sqrt(_H))
    h = _silu(h)
    h = _tdot(w3_ref[...], h) * (1.0 / math.sqrt(_H))
    h = _silu(h) * eat_ref[...]
    o_ref[...] = _tdot(h, w4_ref[...]) * (1.0 / math.sqrt(_H))


def _edge_mlp(eft, eat, W1, W2, W3, W4):
    blk = 4096
    return pl.pallas_call(
        _edge_mlp_body,
        grid=(_EPAD // blk,),
        in_specs=[
            pl.BlockSpec((_R, blk), lambda i: (0, i)),
            pl.BlockSpec((1, blk), lambda i: (0, i)),
            pl.BlockSpec((_R, _H), lambda i: (0, 0)),
            pl.BlockSpec((_H, _H), lambda i: (0, 0)),
            pl.BlockSpec((_H, _H), lambda i: (0, 0)),
            pl.BlockSpec((_H, _D), lambda i: (0, 0)),
        ],
        out_specs=pl.BlockSpec((blk, _D), lambda i: (i, 0)),
        out_shape=jax.ShapeDtypeStruct((_EPAD, _D), jnp.float32),
    )(eft, eat, W1, W2, W3, W4)


def _sc_gather_scatter(x, w, src2, dst2):
    mesh = plsc.VectorSubcoreMesh(core_axis_name="c", subcore_axis_name="s",
                                  num_cores=_NC, num_subcores=_NS)

    @functools.partial(
        pl.kernel,
        out_type=jax.ShapeDtypeStruct((_NC, _NPAD, _D), jnp.float32),
        mesh=mesh,
        scratch_types=[
            pltpu.VMEM((_GRP, _CHUNK), jnp.int32),
            pltpu.VMEM((_GRP, _CHUNK), jnp.int32),
            pltpu.VMEM((_CHUNK, _D), jnp.float32),
            pltpu.VMEM((_CHUNK, _D), jnp.float32),
            pltpu.VMEM((_CHUNK, _D), jnp.float32),
            pltpu.VMEM((_CHUNK, _D), jnp.float32),
            pltpu.VMEM_SHARED((_NPAD, _D), jnp.float32),
            pltpu.SemaphoreType.DMA,
            pltpu.SemaphoreType.DMA,
            pltpu.SemaphoreType.DMA,
            pltpu.SemaphoreType.DMA,
        ],
    )
    def k(x_hbm, w_hbm, src_hbm, dst_hbm, out_hbm,
          src_v, dst_v, rows0, rows1, wrow0, wrow1, acc,
          sg0, sg1, sw0, sw1):
        cid = lax.axis_index("c")
        sid = lax.axis_index("s")
        wid = cid * _NS + sid
        rows = (rows0, rows1)
        wrow = (wrow0, wrow1)
        sg = (sg0, sg1)
        sw = (sw0, sw1)

        def zrow(i, carry):
            for k8 in range(_D // 16):
                rows0[i, pl.ds(k8 * 16, 16)] = jnp.zeros((16,), jnp.float32)
            return carry
        lax.fori_loop(0, _CHUNK, zrow, 0)
        base_row = sid * _RPT
        for kk in range(_RPT // _CHUNK):
            pltpu.sync_copy(rows0, acc.at[pl.ds(base_row + kk * _CHUNK, _CHUNK)])
        plsc.subcore_barrier()

        def issue(h, c, b):
            ebase = wid * _EPW + h * _GRP * _CHUNK
            pltpu.async_copy(x_hbm.at[src_v.at[c]], rows[b], sg[b])
            pltpu.async_copy(w_hbm.at[pl.ds(ebase + c * _CHUNK, _CHUNK)],
                             wrow[b], sw[b])

        def wait(b):
            pltpu.make_async_copy(x_hbm.at[src_v.at[0]], rows[b], sg[b]).wait()
            pltpu.make_async_copy(w_hbm.at[pl.ds(0, _CHUNK)], wrow[b], sw[b]).wait()

        def process(c, b):
            @plsc.parallel_loop(0, _CHUNK, unroll=4)
            def mul(i):
                for k8 in range(_D // 16):
                    sl = pl.ds(k8 * 16, 16)
                    rows[b][i, sl] = rows[b][i, sl] * wrow[b][i, sl]
            pltpu.sync_copy(rows[b], acc.at[dst_v.at[c]], add=True)

        for h in range(_NGRP):
            pltpu.sync_copy(src_hbm.at[pl.ds(wid * _CPW + h * _GRP, _GRP)], src_v)
            pltpu.sync_copy(dst_hbm.at[pl.ds(wid * _CPW + h * _GRP, _GRP)], dst_v)
            issue(h, 0, 0)

            def pair(i, carry, h=h):
                c0 = 2 * i
                issue(h, c0 + 1, 1)
                wait(0)
                process(c0, 0)

                @pl.when(i < _GRP // 2 - 1)
                def _():
                    issue(h, c0 + 2, 0)
                wait(1)
                process(c0 + 1, 1)
                return carry
            lax.fori_loop(0, _GRP // 2, pair, 0)
        plsc.subcore_barrier()

        for kk in range(_RPT // _CHUNK):
            r0 = base_row + kk * _CHUNK
            pltpu.sync_copy(acc.at[pl.ds(r0, _CHUNK)], rows0)
            pltpu.sync_copy(rows0, out_hbm.at[cid].at[pl.ds(r0, _CHUNK)])

    return k(x, w, src2, dst2)


def _finish_body(p_ref, na_ref, wlin_ref, wskip_ref, o_ref):
    msg = (p_ref[0] + p_ref[1])
    msg = jnp.dot(msg, wlin_ref[...],
                  preferred_element_type=jnp.float32) * (1.0 / (math.sqrt(_D) * _AVG))
    acc = jnp.zeros_like(o_ref)
    for v in range(_A):
        acc = acc + na_ref[:, v:v + 1] * jnp.dot(
            msg, wskip_ref[v], preferred_element_type=jnp.float32)
    o_ref[...] = acc * (1.0 / math.sqrt(float(_D * _A)))


def _finish(partial, node_attrs, W_lin, W_skip_t):
    blk = 1000
    return pl.pallas_call(
        _finish_body,
        grid=(_N // blk,),
        in_specs=[
            pl.BlockSpec((_NC, blk, _D), lambda i: (0, i, 0)),
            pl.BlockSpec((blk, _A), lambda i: (i, 0)),
            pl.BlockSpec((_D, _D), lambda i: (0, 0)),
            pl.BlockSpec((_A, _D, _D), lambda i: (0, 0, 0)),
        ],
        out_specs=pl.BlockSpec((blk, _D), lambda i: (i, 0)),
        out_shape=jax.ShapeDtypeStruct((_N, _D), jnp.float32),
    )(partial, node_attrs, W_lin, W_skip_t)


def kernel(node_attrs, node_feats, edge_attrs, edge_feats, edge_index,
           W_up, W1, W2, W3, W4, W_lin, W_skip):
    pad = _EPAD - _E
    eft = jnp.pad(edge_feats.T, ((0, 0), (0, pad)))
    eat = jnp.pad(edge_attrs.T, ((0, 0), (0, pad)))
    src2 = jnp.pad(edge_index[0], (0, pad)).reshape(_EPAD // _CHUNK, _CHUNK)
    dst2 = jnp.pad(edge_index[1], (0, pad)).reshape(_EPAD // _CHUNK, _CHUNK)

    x = _node_up(node_feats, W_up)
    w = _edge_mlp(eft, eat, W1, W2, W3, W4)
    partial = _sc_gather_scatter(x, w, src2, dst2)
    out = _finish(partial, node_attrs, W_lin, jnp.transpose(W_skip, (1, 0, 2)))
    return out.reshape(_N, _D, 1)
